# baseline (speedup 1.0000x reference)
import functools

import jax
import jax.numpy as jnp
from jax import lax
from jax.experimental import pallas as pl
from jax.experimental.pallas import tpu as pltpu
from jax.experimental.pallas import tpu_sc as plsc

D = 128
NC, NS = 2, 16
NW = NC * NS
EB = 128

_mesh = functools.partial(
    plsc.VectorSubcoreMesh, core_axis_name="c", subcore_axis_name="s",
    num_cores=NC, num_subcores=NS)


def _fill(ref, n, value, dtype):
    v = jnp.full((16,), value, dtype)
    for i in range(n // 16):
        ref[pl.ds(i * 16, 16)] = v


def _deg_call(dst2, npad, nb):
    rpt = npad // NS

    def body(dst_hbm, degp_hbm, didx_v, ones_v, zero_v, deg_sh, dsem):
        c = lax.axis_index("c")
        s = lax.axis_index("s")
        wid = c * NS + s
        pltpu.sync_copy(dst_hbm.at[wid], didx_v)
        _fill(ones_v, EB, 1.0, jnp.float32)
        _fill(zero_v, rpt, 0.0, jnp.float32)
        r0 = s * rpt
        pltpu.sync_copy(zero_v, deg_sh.at[pl.ds(r0, rpt)])
        plsc.subcore_barrier()

        def step(i, _):
            pltpu.async_copy(ones_v, deg_sh.at[didx_v.at[i]], dsem, add=True)
            return 0

        lax.fori_loop(0, nb, step, 0)
        pltpu.make_async_copy(dst_hbm.at[wid], didx_v, dsem).wait()
        plsc.subcore_barrier()
        pltpu.sync_copy(deg_sh.at[pl.ds(r0, rpt)],
                        degp_hbm.at[c, pl.ds(r0, rpt)])

    return pl.kernel(
        body,
        out_type=jax.ShapeDtypeStruct((NC, npad), jnp.float32),
        mesh=_mesh(),
        scratch_types=[
            pltpu.VMEM((nb, EB), jnp.int32),
            pltpu.VMEM((EB,), jnp.float32),
            pltpu.VMEM((rpt,), jnp.float32),
            pltpu.VMEM_SHARED((npad,), jnp.float32),
            pltpu.SemaphoreType.DMA,
        ],
    )(dst2)


def _agg_call(src2, dst2, y, npad, nb):
    rpt = npad // NS

    nch = 2
    chb = nb // nch

    def body(src_hbm, dst_hbm, y_hbm, out_hbm, sidx_v, didx_v, rows_a, rows_b,
             acc_sh, gsa, gsb):
        c = lax.axis_index("c")
        s = lax.axis_index("s")
        wid = c * NS + s
        r0 = s * rpt

        def run_chunk(first):
            pltpu.async_copy(y_hbm.at[sidx_v.at[0]], rows_a, gsa)
            pltpu.async_copy(y_hbm.at[sidx_v.at[1]], rows_b, gsb)
            if first:
                pltpu.sync_copy(y_hbm.at[pl.ds(r0, rpt)],
                                acc_sh.at[pl.ds(r0, rpt)])
                plsc.subcore_barrier()

            def step(i, _):
                b0 = 2 * i
                pltpu.make_async_copy(
                    y_hbm.at[sidx_v.at[b0]], rows_a, gsa).wait()
                pltpu.sync_copy(rows_a, acc_sh.at[didx_v.at[b0]], add=True)

                @pl.when(b0 + 2 < chb)
                def _():
                    pltpu.async_copy(y_hbm.at[sidx_v.at[b0 + 2]], rows_a, gsa)

                pltpu.make_async_copy(
                    y_hbm.at[sidx_v.at[b0 + 1]], rows_b, gsb).wait()
                pltpu.sync_copy(rows_b, acc_sh.at[didx_v.at[b0 + 1]],
                                add=True)

                @pl.when(b0 + 3 < chb)
                def _():
                    pltpu.async_copy(y_hbm.at[sidx_v.at[b0 + 3]], rows_b, gsb)

                return 0

            lax.fori_loop(0, chb // 2, step, 0)

        for ch in range(nch):
            pltpu.sync_copy(src_hbm.at[wid, pl.ds(ch * chb, chb)], sidx_v)
            pltpu.sync_copy(dst_hbm.at[wid, pl.ds(ch * chb, chb)], didx_v)
            run_chunk(first=(ch == 0))

        plsc.subcore_barrier()
        pltpu.sync_copy(acc_sh.at[pl.ds(r0, rpt)],
                        out_hbm.at[c, pl.ds(r0, rpt)])

    return pl.kernel(
        body,
        out_type=jax.ShapeDtypeStruct((NC, npad, D), jnp.float32),
        mesh=_mesh(),
        scratch_types=[
            pltpu.VMEM((chb, EB), jnp.int32),
            pltpu.VMEM((chb, EB), jnp.int32),
            pltpu.VMEM((EB, D), jnp.float32),
            pltpu.VMEM((EB, D), jnp.float32),
            pltpu.VMEM_SHARED((npad, D), jnp.float32),
            pltpu.SemaphoreType.DMA,
            pltpu.SemaphoreType.DMA,
        ],
    )(src2, dst2, y)


def _y_call(xp, W, degp, npad):
    rb = 1024
    grid = npad // rb

    def body(x_ref, w_ref, degp_ref, y_ref):
        deg = degp_ref[0, :] + degp_ref[1, :] + 1.0
        dinv = lax.rsqrt(deg)
        xw = jnp.dot(x_ref[...], w_ref[...],
                     preferred_element_type=jnp.float32)
        y_ref[...] = xw * dinv[:, None]

    return pl.pallas_call(
        body,
        out_shape=jax.ShapeDtypeStruct((npad, D), jnp.float32),
        grid=(grid,),
        in_specs=[
            pl.BlockSpec((rb, D), lambda j: (j, 0)),
            pl.BlockSpec((D, D), lambda j: (0, 0)),
            pl.BlockSpec((NC, rb), lambda j: (0, j)),
        ],
        out_specs=pl.BlockSpec((rb, D), lambda j: (j, 0)),
    )(xp, W, degp)


def _final_call(P, y, degp, b2, npad):
    rb = 1024
    grid = npad // rb

    def body(p_ref, y_ref, degp_ref, b_ref, o_ref):
        deg = degp_ref[0, :] + degp_ref[1, :] + 1.0
        dinv = lax.rsqrt(deg)
        ssum = p_ref[0, :, :] + p_ref[1, :, :] - y_ref[...]
        o_ref[...] = jnp.maximum(ssum * dinv[:, None] + b_ref[0, :][None, :],
                                 0.0)

    return pl.pallas_call(
        body,
        out_shape=jax.ShapeDtypeStruct((npad, D), jnp.float32),
        grid=(grid,),
        in_specs=[
            pl.BlockSpec((NC, rb, D), lambda j: (0, j, 0)),
            pl.BlockSpec((rb, D), lambda j: (j, 0)),
            pl.BlockSpec((NC, rb), lambda j: (0, j)),
            pl.BlockSpec((1, D), lambda j: (0, 0)),
        ],
        out_specs=pl.BlockSpec((rb, D), lambda j: (j, 0)),
    )(P, y, degp, b2)


def kernel(x, edge_index, W, b):
    n = x.shape[0]
    e = edge_index.shape[1]
    npad = ((n + 1024) // 1024) * 1024
    ept = ((e + NW - 1) // NW + 4 * EB - 1) // (4 * EB) * (4 * EB)
    nb = ept // EB
    epad = ept * NW

    src = edge_index[0].astype(jnp.int32)
    dst = edge_index[1].astype(jnp.int32)
    src2 = jnp.concatenate(
        [src, jnp.zeros((epad - e,), jnp.int32)]).reshape(NW, nb, EB)
    dst2 = jnp.concatenate(
        [dst, jnp.full((epad - e,), n, jnp.int32)]).reshape(NW, nb, EB)
    xp = jnp.pad(x, ((0, npad - n), (0, 0)))

    degp = _deg_call(dst2, npad, nb)
    y = _y_call(xp, W, degp, npad)
    P = _agg_call(src2, dst2, y, npad, nb)
    return _final_call(P, y, degp, b.reshape(1, D), npad)[:n]


# speedup vs baseline: 2.2039x; 2.2039x over previous
import functools

import jax
import jax.numpy as jnp
from jax import lax
from jax.experimental import pallas as pl
from jax.experimental.pallas import tpu as pltpu
from jax.experimental.pallas import tpu_sc as plsc

D = 128
NC, NS = 2, 16
NW = NC * NS
EB = 128

_mesh = functools.partial(
    plsc.VectorSubcoreMesh, core_axis_name="c", subcore_axis_name="s",
    num_cores=NC, num_subcores=NS)


def _fill(ref, n, value, dtype):
    v = jnp.full((16,), value, dtype)
    for i in range(n // 16):
        ref[pl.ds(i * 16, 16)] = v


def _deg_call(dst2, npad, nb):
    rpt = npad // NS

    def body(dst_hbm, degp_hbm, didx_v, ones_v, zero_v, deg_sh, dsem):
        c = lax.axis_index("c")
        s = lax.axis_index("s")
        wid = c * NS + s
        pltpu.sync_copy(dst_hbm.at[wid], didx_v)
        _fill(ones_v, EB, 1.0, jnp.float32)
        _fill(zero_v, rpt, 0.0, jnp.float32)
        r0 = s * rpt
        pltpu.sync_copy(zero_v, deg_sh.at[pl.ds(r0, rpt)])
        plsc.subcore_barrier()

        def step(i, _):
            pltpu.async_copy(ones_v, deg_sh.at[didx_v.at[i]], dsem, add=True)
            return 0

        lax.fori_loop(0, nb, step, 0)
        pltpu.make_async_copy(dst_hbm.at[wid], didx_v, dsem).wait()
        plsc.subcore_barrier()
        pltpu.sync_copy(deg_sh.at[pl.ds(r0, rpt)],
                        degp_hbm.at[c, pl.ds(r0, rpt)])

    return pl.kernel(
        body,
        out_type=jax.ShapeDtypeStruct((NC, npad), jnp.float32),
        mesh=_mesh(),
        scratch_types=[
            pltpu.VMEM((nb, EB), jnp.int32),
            pltpu.VMEM((EB,), jnp.float32),
            pltpu.VMEM((rpt,), jnp.float32),
            pltpu.VMEM_SHARED((npad,), jnp.float32),
            pltpu.SemaphoreType.DMA,
        ],
    )(dst2)


def _agg_call(src2, dst2, y, npad, nb):
    rpt = npad // NS
    nch = 2
    chb = nb // nch

    def body(src_hbm, dst_hbm, y_hbm, out_hbm, sidx_v, didx_v, rows_a,
             acc_sh):
        c = lax.axis_index("c")
        s = lax.axis_index("s")
        wid = c * NS + s
        r0 = s * rpt

        pltpu.sync_copy(y_hbm.at[pl.ds(r0, rpt)], acc_sh.at[pl.ds(r0, rpt)])
        plsc.subcore_barrier()

        def step(i, _):
            pltpu.sync_copy(y_hbm.at[sidx_v.at[i]], rows_a)
            pltpu.sync_copy(rows_a, acc_sh.at[didx_v.at[i]], add=True)
            return 0

        for ch in range(nch):
            pltpu.sync_copy(src_hbm.at[wid, pl.ds(ch * chb, chb)], sidx_v)
            pltpu.sync_copy(dst_hbm.at[wid, pl.ds(ch * chb, chb)], didx_v)
            lax.fori_loop(0, chb, step, 0)
        plsc.subcore_barrier()
        pltpu.sync_copy(acc_sh.at[pl.ds(r0, rpt)],
                        out_hbm.at[c, pl.ds(r0, rpt)])

    return pl.kernel(
        body,
        out_type=jax.ShapeDtypeStruct((NC, npad, D), jnp.float32),
        mesh=_mesh(),
        scratch_types=[
            pltpu.VMEM((chb, EB), jnp.int32),
            pltpu.VMEM((chb, EB), jnp.int32),
            pltpu.VMEM((EB, D), jnp.float32),
            pltpu.VMEM_SHARED((npad, D), jnp.float32),
        ],
    )(src2, dst2, y)


def _y_call(xp, W, degp, npad):
    rb = 1024
    grid = npad // rb

    def body(x_ref, w_ref, degp_ref, y_ref):
        deg = degp_ref[0, :] + degp_ref[1, :] + 1.0
        dinv = lax.rsqrt(deg)
        xw = jnp.dot(x_ref[...], w_ref[...],
                     preferred_element_type=jnp.float32)
        y_ref[...] = xw * dinv[:, None]

    return pl.pallas_call(
        body,
        out_shape=jax.ShapeDtypeStruct((npad, D), jnp.float32),
        grid=(grid,),
        in_specs=[
            pl.BlockSpec((rb, D), lambda j: (j, 0)),
            pl.BlockSpec((D, D), lambda j: (0, 0)),
            pl.BlockSpec((NC, rb), lambda j: (0, j)),
        ],
        out_specs=pl.BlockSpec((rb, D), lambda j: (j, 0)),
    )(xp, W, degp)


def _final_call(P, y, degp, b2, npad):
    rb = 1024
    grid = npad // rb

    def body(p_ref, y_ref, degp_ref, b_ref, o_ref):
        deg = degp_ref[0, :] + degp_ref[1, :] + 1.0
        dinv = lax.rsqrt(deg)
        ssum = p_ref[0, :, :] + p_ref[1, :, :] - y_ref[...]
        o_ref[...] = jnp.maximum(
            ssum * dinv[:, None] + b_ref[0, :][None, :], 0.0)

    return pl.pallas_call(
        body,
        out_shape=jax.ShapeDtypeStruct((npad, D), jnp.float32),
        grid=(grid,),
        in_specs=[
            pl.BlockSpec((NC, rb, D), lambda j: (0, j, 0)),
            pl.BlockSpec((rb, D), lambda j: (j, 0)),
            pl.BlockSpec((NC, rb), lambda j: (0, j)),
            pl.BlockSpec((1, D), lambda j: (0, 0)),
        ],
        out_specs=pl.BlockSpec((rb, D), lambda j: (j, 0)),
    )(P, y, degp, b2)


def kernel(x, edge_index, W, b):
    n = x.shape[0]
    e = edge_index.shape[1]
    npad = ((n + 1024) // 1024) * 1024
    ept = ((e + NW - 1) // NW + 4 * EB - 1) // (4 * EB) * (4 * EB)
    nb = ept // EB
    epad = ept * NW

    src = edge_index[0].astype(jnp.int32)
    dst = edge_index[1].astype(jnp.int32)
    pad = jnp.arange(epad - e, dtype=jnp.int32)
    src2 = jnp.concatenate([src, pad % n]).reshape(NW, nb, EB)
    dst2 = jnp.concatenate([dst, n + pad % (npad - n)]).reshape(NW, nb, EB)
    xp = jnp.pad(x, ((0, npad - n), (0, 0)))

    degp = _deg_call(dst2, npad, nb)
    y = _y_call(xp, W, degp, npad)
    P = _agg_call(src2, dst2, y, npad, nb)
    return _final_call(P, y, degp, b.reshape(1, D), npad)[:n]


# speedup vs baseline: 3.0533x; 1.3854x over previous
import functools

import jax
import jax.numpy as jnp
from jax import lax
from jax.experimental import pallas as pl
from jax.experimental.pallas import tpu as pltpu
from jax.experimental.pallas import tpu_sc as plsc

D = 128
NC, NS = 2, 16
NW = NC * NS
EB = 128

_mesh = functools.partial(
    plsc.VectorSubcoreMesh, core_axis_name="c", subcore_axis_name="s",
    num_cores=NC, num_subcores=NS)


def _fill(ref, n, value, dtype):
    v = jnp.full((16,), value, dtype)
    for i in range(n // 16):
        ref[pl.ds(i * 16, 16)] = v


def _deg_call(dst2, npad, nb):
    rpt = npad // NS

    def body(dst_hbm, degp_hbm, didx_v, ones_v, zero_v, deg_sh, dsem):
        c = lax.axis_index("c")
        s = lax.axis_index("s")
        wid = c * NS + s
        pltpu.sync_copy(dst_hbm.at[wid], didx_v)
        _fill(ones_v, EB, 1.0, jnp.float32)
        _fill(zero_v, rpt, 0.0, jnp.float32)
        r0 = s * rpt
        pltpu.sync_copy(zero_v, deg_sh.at[pl.ds(r0, rpt)])
        plsc.subcore_barrier()

        def step(i, _):
            pltpu.async_copy(ones_v, deg_sh.at[didx_v.at[i]], dsem, add=True)
            return 0

        lax.fori_loop(0, nb, step, 0)
        pltpu.make_async_copy(dst_hbm.at[wid], didx_v, dsem).wait()
        plsc.subcore_barrier()
        pltpu.sync_copy(deg_sh.at[pl.ds(r0, rpt)],
                        degp_hbm.at[c, pl.ds(r0, rpt)])

    return pl.kernel(
        body,
        out_type=jax.ShapeDtypeStruct((NC, npad), jnp.float32),
        mesh=_mesh(),
        scratch_types=[
            pltpu.VMEM((nb, EB), jnp.int32),
            pltpu.VMEM((EB,), jnp.float32),
            pltpu.VMEM((rpt,), jnp.float32),
            pltpu.VMEM_SHARED((npad,), jnp.float32),
            pltpu.SemaphoreType.DMA,
        ],
    )(dst2)


def _agg_call(src2, dst2, y, npad, nb):
    rpt = npad // NS
    nch = 2
    chb = nb // nch

    def body(src_hbm, dst_hbm, y_hbm, out_hbm, sidx_v, didx_v, rows_a, rows_b,
             acc_sh, gsa, gsb):
        c = lax.axis_index("c")
        s = lax.axis_index("s")
        wid = c * NS + s
        r0 = s * rpt

        pltpu.sync_copy(y_hbm.at[pl.ds(r0, rpt)], acc_sh.at[pl.ds(r0, rpt)])
        plsc.subcore_barrier()

        def run_chunk():
            pltpu.async_copy(y_hbm.at[sidx_v.at[0]], rows_a, gsa)
            pltpu.async_copy(y_hbm.at[sidx_v.at[1]], rows_b, gsb)

            def step(i, _):
                b0 = 2 * i
                pltpu.make_async_copy(
                    y_hbm.at[sidx_v.at[b0]], rows_a, gsa).wait()
                pltpu.sync_copy(rows_a, acc_sh.at[didx_v.at[b0]], add=True)
                pltpu.async_copy(y_hbm.at[sidx_v.at[b0 + 2]], rows_a, gsa)
                pltpu.make_async_copy(
                    y_hbm.at[sidx_v.at[b0 + 1]], rows_b, gsb).wait()
                pltpu.sync_copy(rows_b, acc_sh.at[didx_v.at[b0 + 1]],
                                add=True)
                pltpu.async_copy(y_hbm.at[sidx_v.at[b0 + 3]], rows_b, gsb)
                return 0

            lax.fori_loop(0, chb // 2 - 1, step, 0)
            pltpu.make_async_copy(
                y_hbm.at[sidx_v.at[chb - 2]], rows_a, gsa).wait()
            pltpu.sync_copy(rows_a, acc_sh.at[didx_v.at[chb - 2]], add=True)
            pltpu.make_async_copy(
                y_hbm.at[sidx_v.at[chb - 1]], rows_b, gsb).wait()
            pltpu.sync_copy(rows_b, acc_sh.at[didx_v.at[chb - 1]], add=True)

        for ch in range(nch):
            pltpu.sync_copy(src_hbm.at[wid, pl.ds(ch * chb, chb)], sidx_v)
            pltpu.sync_copy(dst_hbm.at[wid, pl.ds(ch * chb, chb)], didx_v)
            run_chunk()
        plsc.subcore_barrier()
        pltpu.sync_copy(acc_sh.at[pl.ds(r0, rpt)],
                        out_hbm.at[c, pl.ds(r0, rpt)])

    return pl.kernel(
        body,
        out_type=jax.ShapeDtypeStruct((NC, npad, D), jnp.float32),
        mesh=_mesh(),
        scratch_types=[
            pltpu.VMEM((chb, EB), jnp.int32),
            pltpu.VMEM((chb, EB), jnp.int32),
            pltpu.VMEM((EB, D), jnp.float32),
            pltpu.VMEM((EB, D), jnp.float32),
            pltpu.VMEM_SHARED((npad, D), jnp.float32),
            pltpu.SemaphoreType.DMA,
            pltpu.SemaphoreType.DMA,
        ],
    )(src2, dst2, y)


def _y_call(xp, W, degp, npad):
    rb = 1024
    grid = npad // rb

    def body(x_ref, w_ref, degp_ref, y_ref):
        deg = degp_ref[0, :] + degp_ref[1, :] + 1.0
        dinv = lax.rsqrt(deg)
        xw = jnp.dot(x_ref[...], w_ref[...],
                     preferred_element_type=jnp.float32)
        y_ref[...] = xw * dinv[:, None]

    return pl.pallas_call(
        body,
        out_shape=jax.ShapeDtypeStruct((npad, D), jnp.float32),
        grid=(grid,),
        in_specs=[
            pl.BlockSpec((rb, D), lambda j: (j, 0)),
            pl.BlockSpec((D, D), lambda j: (0, 0)),
            pl.BlockSpec((NC, rb), lambda j: (0, j)),
        ],
        out_specs=pl.BlockSpec((rb, D), lambda j: (j, 0)),
    )(xp, W, degp)


def _final_call(P, y, degp, b2, npad):
    rb = 1024
    grid = npad // rb

    def body(p_ref, y_ref, degp_ref, b_ref, o_ref):
        deg = degp_ref[0, :] + degp_ref[1, :] + 1.0
        dinv = lax.rsqrt(deg)
        ssum = p_ref[0, :, :] + p_ref[1, :, :] - y_ref[...]
        o_ref[...] = jnp.maximum(
            ssum * dinv[:, None] + b_ref[0, :][None, :], 0.0)

    return pl.pallas_call(
        body,
        out_shape=jax.ShapeDtypeStruct((npad, D), jnp.float32),
        grid=(grid,),
        in_specs=[
            pl.BlockSpec((NC, rb, D), lambda j: (0, j, 0)),
            pl.BlockSpec((rb, D), lambda j: (j, 0)),
            pl.BlockSpec((NC, rb), lambda j: (0, j)),
            pl.BlockSpec((1, D), lambda j: (0, 0)),
        ],
        out_specs=pl.BlockSpec((rb, D), lambda j: (j, 0)),
    )(P, y, degp, b2)


def kernel(x, edge_index, W, b):
    n = x.shape[0]
    e = edge_index.shape[1]
    npad = ((n + 1024) // 1024) * 1024
    ept = ((e + NW - 1) // NW + 4 * EB - 1) // (4 * EB) * (4 * EB)
    nb = ept // EB
    epad = ept * NW

    src = edge_index[0].astype(jnp.int32)
    dst = edge_index[1].astype(jnp.int32)
    pad = jnp.arange(epad - e, dtype=jnp.int32)
    src2 = jnp.concatenate([src, pad % n]).reshape(NW, nb, EB)
    dst2 = jnp.concatenate([dst, n + pad % (npad - n)]).reshape(NW, nb, EB)
    xp = jnp.pad(x, ((0, npad - n), (0, 0)))

    degp = _deg_call(dst2, npad, nb)
    y = _y_call(xp, W, degp, npad)
    P = _agg_call(src2, dst2, y, npad, nb)
    return _final_call(P, y, degp, b.reshape(1, D), npad)[:n]


# speedup vs baseline: 3.0560x; 1.0009x over previous
import functools

import jax
import jax.numpy as jnp
from jax import lax
from jax.experimental import pallas as pl
from jax.experimental.pallas import tpu as pltpu
from jax.experimental.pallas import tpu_sc as plsc

D = 128
NC, NS = 2, 16
NW = NC * NS
EB = 128

_mesh = functools.partial(
    plsc.VectorSubcoreMesh, core_axis_name="c", subcore_axis_name="s",
    num_cores=NC, num_subcores=NS)


def _fill(ref, n, value, dtype):
    v = jnp.full((16,), value, dtype)
    for i in range(n // 16):
        ref[pl.ds(i * 16, 16)] = v


def _deg_call(dst2, npad, nb):
    rpt = npad // NS

    def body(dst_hbm, degp_hbm, didx_v, ones_v, zero_v, deg_sh, dsem):
        c = lax.axis_index("c")
        s = lax.axis_index("s")
        wid = c * NS + s
        pltpu.sync_copy(dst_hbm.at[wid], didx_v)
        _fill(ones_v, EB, 1.0, jnp.float32)
        _fill(zero_v, rpt, 0.0, jnp.float32)
        r0 = s * rpt
        pltpu.sync_copy(zero_v, deg_sh.at[pl.ds(r0, rpt)])
        plsc.subcore_barrier()

        def step(i, _):
            pltpu.async_copy(ones_v, deg_sh.at[didx_v.at[i]], dsem, add=True)
            return 0

        lax.fori_loop(0, nb, step, 0)
        pltpu.make_async_copy(dst_hbm.at[wid], didx_v, dsem).wait()
        plsc.subcore_barrier()
        pltpu.sync_copy(deg_sh.at[pl.ds(r0, rpt)],
                        degp_hbm.at[c, pl.ds(r0, rpt)])

    return pl.kernel(
        body,
        out_type=jax.ShapeDtypeStruct((NC, npad), jnp.float32),
        mesh=_mesh(),
        scratch_types=[
            pltpu.VMEM((nb, EB), jnp.int32),
            pltpu.VMEM((EB,), jnp.float32),
            pltpu.VMEM((rpt,), jnp.float32),
            pltpu.VMEM_SHARED((npad,), jnp.float32),
            pltpu.SemaphoreType.DMA,
        ],
    )(dst2)


def _agg_call(src2, dst2, y, npad, nb):
    rpt = npad // NS
    nch = 2
    chb = nb // nch
    NBUF = 2

    def body(src_hbm, dst_hbm, y_hbm, out_hbm, sidx_v, didx_v,
             r0v, r1v, acc_sh, g0, g1):
        rows = [r0v, r1v]
        sems = [g0, g1]
        c = lax.axis_index("c")
        s = lax.axis_index("s")
        wid = c * NS + s
        r0 = s * rpt

        pltpu.sync_copy(y_hbm.at[pl.ds(r0, rpt)], acc_sh.at[pl.ds(r0, rpt)])
        plsc.subcore_barrier()

        def run_chunk():
            for j in range(NBUF):
                pltpu.async_copy(y_hbm.at[sidx_v.at[j]], rows[j], sems[j])

            def step(i, _):
                b0 = NBUF * i
                for j in range(NBUF):
                    pltpu.make_async_copy(
                        y_hbm.at[sidx_v.at[b0 + j]], rows[j], sems[j]).wait()
                    pltpu.sync_copy(rows[j], acc_sh.at[didx_v.at[b0 + j]],
                                    add=True)
                    pltpu.async_copy(
                        y_hbm.at[sidx_v.at[b0 + NBUF + j]], rows[j], sems[j])
                return 0

            lax.fori_loop(0, chb // NBUF - 1, step, 0)
            for j in range(NBUF):
                b = chb - NBUF + j
                pltpu.make_async_copy(
                    y_hbm.at[sidx_v.at[b]], rows[j], sems[j]).wait()
                pltpu.sync_copy(rows[j], acc_sh.at[didx_v.at[b]], add=True)

        for ch in range(nch):
            pltpu.sync_copy(src_hbm.at[wid, pl.ds(ch * chb, chb)], sidx_v)
            pltpu.sync_copy(dst_hbm.at[wid, pl.ds(ch * chb, chb)], didx_v)
            run_chunk()
        plsc.subcore_barrier()
        pltpu.sync_copy(acc_sh.at[pl.ds(r0, rpt)],
                        out_hbm.at[c, pl.ds(r0, rpt)])

    return pl.kernel(
        body,
        out_type=jax.ShapeDtypeStruct((NC, npad, D), jnp.float32),
        mesh=_mesh(),
        scratch_types=[
            pltpu.VMEM((chb, EB), jnp.int32),
            pltpu.VMEM((chb, EB), jnp.int32),
            pltpu.VMEM((EB, D), jnp.float32),
            pltpu.VMEM((EB, D), jnp.float32),
            pltpu.VMEM_SHARED((npad, D), jnp.float32),
            pltpu.SemaphoreType.DMA,
            pltpu.SemaphoreType.DMA,
        ],
    )(src2, dst2, y)


def _y_call(xp, W, degp, npad):
    rb = 1024
    grid = npad // rb

    def body(x_ref, w_ref, degp_ref, y_ref):
        deg = degp_ref[0, :] + degp_ref[1, :] + 1.0
        dinv = lax.rsqrt(deg)
        xw = jnp.dot(x_ref[...], w_ref[...],
                     preferred_element_type=jnp.float32)
        y_ref[...] = xw * dinv[:, None]

    return pl.pallas_call(
        body,
        out_shape=jax.ShapeDtypeStruct((npad, D), jnp.float32),
        grid=(grid,),
        in_specs=[
            pl.BlockSpec((rb, D), lambda j: (j, 0)),
            pl.BlockSpec((D, D), lambda j: (0, 0)),
            pl.BlockSpec((NC, rb), lambda j: (0, j)),
        ],
        out_specs=pl.BlockSpec((rb, D), lambda j: (j, 0)),
    )(xp, W, degp)


def _final_call(P, y, degp, b2, npad):
    rb = 1024
    grid = npad // rb

    def body(p_ref, y_ref, degp_ref, b_ref, o_ref):
        deg = degp_ref[0, :] + degp_ref[1, :] + 1.0
        dinv = lax.rsqrt(deg)
        ssum = p_ref[0, :, :] + p_ref[1, :, :] - y_ref[...]
        o_ref[...] = jnp.maximum(
            ssum * dinv[:, None] + b_ref[0, :][None, :], 0.0)

    return pl.pallas_call(
        body,
        out_shape=jax.ShapeDtypeStruct((npad, D), jnp.float32),
        grid=(grid,),
        in_specs=[
            pl.BlockSpec((NC, rb, D), lambda j: (0, j, 0)),
            pl.BlockSpec((rb, D), lambda j: (j, 0)),
            pl.BlockSpec((NC, rb), lambda j: (0, j)),
            pl.BlockSpec((1, D), lambda j: (0, 0)),
        ],
        out_specs=pl.BlockSpec((rb, D), lambda j: (j, 0)),
    )(P, y, degp, b2)


def kernel(x, edge_index, W, b):
    n = x.shape[0]
    e = edge_index.shape[1]
    npad = ((n + 1024) // 1024) * 1024
    ept = ((e + NW - 1) // NW + 16 * EB - 1) // (16 * EB) * (16 * EB)
    nb = ept // EB
    epad = ept * NW

    src = edge_index[0].astype(jnp.int32)
    dst = edge_index[1].astype(jnp.int32)
    pad = jnp.arange(epad - e, dtype=jnp.int32)
    src2 = jnp.concatenate([src, pad % n]).reshape(NW, nb, EB)
    dst2 = jnp.concatenate([dst, n + pad % (npad - n)]).reshape(NW, nb, EB)
    xp = jnp.pad(x, ((0, npad - n), (0, 0)))

    degp = _deg_call(dst2, npad, nb)
    y = _y_call(xp, W, degp, npad)
    P = _agg_call(src2, dst2, y, npad, nb)
    return _final_call(P, y, degp, b.reshape(1, D), npad)[:n]
